# Initial kernel scaffold; baseline (speedup 1.0000x reference)
import jax
import jax.numpy as jnp
from jax.experimental import pallas as pl


def kernel(images, conv_w, conv_b, lin_w, lin_b):
    raise NotImplementedError("write your pallas kernel here")


# speedup vs baseline: 2.8123x; 2.8123x over previous
import functools

import jax
import jax.numpy as jnp
from jax.experimental import pallas as pl
from jax.experimental.pallas import tpu as pltpu


def _conv_pool_body(x_ref, w_ref, b_ref, o_ref, *, H, W, C, K, HD):
    HW = H * W
    pad = (K - 1) // 2
    MARG = 8 + pad * W
    xv = x_ref[0]
    buf = jnp.concatenate(
        [jnp.zeros((MARG, C), xv.dtype), xv, jnp.zeros((MARG, C), xv.dtype)],
        axis=0,
    )
    w_idx = jax.lax.broadcasted_iota(jnp.int32, (HW, 1), 0) % W
    taps = []
    for kh in range(K):
        for kw in range(K):
            dw = kw - pad
            start = 8 + kh * W + dw
            t = buf[start : start + HW]
            if dw != 0:
                valid = (w_idx + dw >= 0) & (w_idx + dw < W)
                t = jnp.where(valid, t, jnp.zeros((), t.dtype))
            taps.append(t)
    patches = jnp.concatenate(taps, axis=1)
    a = jnp.dot(patches, w_ref[...], preferred_element_type=jnp.float32)
    a = a.reshape(HW // 2, 2, HD)
    a = jnp.max(a, axis=1)
    Wo = W // 2
    a = a.reshape(H // 2, 2, Wo, HD)
    a = jnp.max(a, axis=1)
    a = a.reshape((H // 2) * Wo, HD)
    o_ref[0] = jnp.maximum(a + b_ref[...], 0.0).astype(o_ref.dtype)


def _head_body(x_ref, w_ref, b_ref, o_ref):
    o_ref[...] = (
        jnp.dot(x_ref[...], w_ref[...], preferred_element_type=jnp.float32)
        + b_ref[...]
    )


@jax.jit
def kernel(images, conv_w, conv_b, lin_w, lin_b):
    N, C, H, W = images.shape
    HD = conv_w.shape[0]
    K = conv_w.shape[2]
    NCLS = lin_w.shape[0]
    Ho, Wo = H // 2, W // 2
    HWo = Ho * Wo
    KKC = K * K * C

    x = (
        jnp.transpose(images, (0, 2, 3, 1))
        .reshape(N, H * W, C)
        .astype(jnp.bfloat16)
    )
    w_mat = (
        jnp.transpose(conv_w, (2, 3, 1, 0)).reshape(KKC, HD).astype(jnp.bfloat16)
    )
    b_mat = conv_b.reshape(1, HD)

    pooled = pl.pallas_call(
        functools.partial(_conv_pool_body, H=H, W=W, C=C, K=K, HD=HD),
        out_shape=jax.ShapeDtypeStruct((N, HWo, HD), jnp.bfloat16),
        grid=(N,),
        in_specs=[
            pl.BlockSpec((1, H * W, C), lambda n: (n, 0, 0)),
            pl.BlockSpec((KKC, HD), lambda n: (0, 0)),
            pl.BlockSpec((1, HD), lambda n: (0, 0)),
        ],
        out_specs=pl.BlockSpec((1, HWo, HD), lambda n: (n, 0, 0)),
        compiler_params=pltpu.CompilerParams(
            dimension_semantics=("parallel",)
        ),
    )(x, w_mat, b_mat)

    feats = pooled.reshape(N, HWo * HD)
    wl = (
        lin_w.reshape(NCLS, HD, Ho, Wo)
        .transpose(2, 3, 1, 0)
        .reshape(HWo * HD, NCLS)
        .astype(jnp.bfloat16)
    )
    bl = lin_b.reshape(1, NCLS)

    logits = pl.pallas_call(
        _head_body,
        out_shape=jax.ShapeDtypeStruct((N, NCLS), jnp.float32),
    )(feats, wl, bl)
    return logits


# speedup vs baseline: 3.5484x; 1.2618x over previous
import functools

import jax
import jax.numpy as jnp
from jax.experimental import pallas as pl
from jax.experimental.pallas import tpu as pltpu


def _conv_pool_body(x_ref, w_ref, b_ref, o_ref, *, H, W, G, GC, K, GHD):
    HW = H * W
    pad = (K - 1) // 2
    xv = x_ref[0]
    z = jnp.zeros((pad * W, GC), xv.dtype)
    xp = jnp.concatenate([z, xv, z], axis=0)
    PW = (H + 2 * pad) * W
    w_idx = jax.lax.broadcasted_iota(jnp.int32, (PW, 1), 0) % W
    z1 = jnp.zeros((1, GC), xv.dtype)
    variants = {0: xp}
    for dw in range(-pad, pad + 1):
        if dw == 0:
            continue
        if dw < 0:
            v = jnp.concatenate([jnp.tile(z1, (-dw, 1)), xp[:dw]], axis=0)
        else:
            v = jnp.concatenate([xp[dw:], jnp.tile(z1, (dw, 1))], axis=0)
        valid = (w_idx + dw >= 0) & (w_idx + dw < W)
        variants[dw] = jnp.where(valid, v, jnp.zeros((), v.dtype))
    taps = [
        variants[kw - pad][kh * W : kh * W + HW]
        for kh in range(K)
        for kw in range(K)
    ]
    patches = jnp.concatenate(taps, axis=1)
    a = jnp.dot(patches, w_ref[...], preferred_element_type=jnp.float32)
    a = jnp.max(a.reshape(HW // 2, 2, GHD), axis=1)
    a = jnp.max(a.reshape(H // 2, 2, (W // 2), GHD), axis=1)
    a = a.reshape((H // 2) * (W // 2), GHD)
    o_ref[0] = jnp.maximum(a + b_ref[...], 0.0).astype(o_ref.dtype)


def _head_body(x_ref, w_ref, b_ref, o_ref):
    o_ref[...] = (
        jnp.dot(x_ref[...], w_ref[...], preferred_element_type=jnp.float32)
        + b_ref[...]
    )


@jax.jit
def kernel(images, conv_w, conv_b, lin_w, lin_b):
    N, C, H, W = images.shape
    HD = conv_w.shape[0]
    K = conv_w.shape[2]
    NCLS = lin_w.shape[0]
    Ho, Wo = H // 2, W // 2
    HWo = Ho * Wo
    G = 4
    NB = N // G

    x = (
        images.reshape(NB, G, C, H * W)
        .transpose(0, 3, 1, 2)
        .reshape(NB, H * W, G * C)
        .astype(jnp.bfloat16)
    )
    wt = jnp.transpose(conv_w, (2, 3, 1, 0)).reshape(K * K, C, HD)
    w4 = jnp.einsum("gh,tcd->tgchd", jnp.eye(G, dtype=wt.dtype), wt)
    w4 = w4.reshape(K * K * G * C, G * HD).astype(jnp.bfloat16)
    b4 = jnp.tile(conv_b, G).reshape(1, G * HD)

    pooled = pl.pallas_call(
        functools.partial(
            _conv_pool_body, H=H, W=W, G=G, GC=G * C, K=K, GHD=G * HD
        ),
        out_shape=jax.ShapeDtypeStruct((NB, HWo, G * HD), jnp.bfloat16),
        grid=(NB,),
        in_specs=[
            pl.BlockSpec((1, H * W, G * C), lambda n: (n, 0, 0)),
            pl.BlockSpec((K * K * G * C, G * HD), lambda n: (0, 0)),
            pl.BlockSpec((1, G * HD), lambda n: (0, 0)),
        ],
        out_specs=pl.BlockSpec((1, HWo, G * HD), lambda n: (n, 0, 0)),
        compiler_params=pltpu.CompilerParams(
            dimension_semantics=("parallel",)
        ),
    )(x, w4, b4)

    feats = (
        pooled.reshape(NB, HWo, G, HD)
        .transpose(0, 2, 1, 3)
        .reshape(N, HWo * HD)
    )
    wl = (
        lin_w.reshape(NCLS, HD, Ho, Wo)
        .transpose(2, 3, 1, 0)
        .reshape(HWo * HD, NCLS)
        .astype(jnp.bfloat16)
    )
    bl = lin_b.reshape(1, NCLS)

    logits = pl.pallas_call(
        _head_body,
        out_shape=jax.ShapeDtypeStruct((N, NCLS), jnp.float32),
    )(feats, wl, bl)
    return logits


# speedup vs baseline: 4.3273x; 1.2195x over previous
import functools

import jax
import jax.numpy as jnp
from jax.experimental import pallas as pl
from jax.experimental.pallas import tpu as pltpu


def _conv_pool_body(x_ref, w_ref, b_ref, s_ref, o_ref, *, H, W, GC, K, GHD):
    HW = H * W
    pad = (K - 1) // 2
    xv = x_ref[0]
    z = jnp.zeros((pad * W, GC), xv.dtype)
    xp = jnp.concatenate([z, xv, z], axis=0)
    PW = (H + 2 * pad) * W
    w_idx = jax.lax.broadcasted_iota(jnp.int32, (PW, 1), 0) % W
    z1 = jnp.zeros((1, GC), xv.dtype)
    variants = {0: xp}
    for dw in range(-pad, pad + 1):
        if dw == 0:
            continue
        if dw < 0:
            v = jnp.concatenate([jnp.tile(z1, (-dw, 1)), xp[:dw]], axis=0)
        else:
            v = jnp.concatenate([xp[dw:], jnp.tile(z1, (dw, 1))], axis=0)
        valid = (w_idx + dw >= 0) & (w_idx + dw < W)
        variants[dw] = jnp.where(valid, v, jnp.zeros((), v.dtype))
    taps = [
        variants[kw - pad][kh * W : kh * W + HW]
        for kh in range(K)
        for kw in range(K)
    ]
    patches = jnp.concatenate(taps, axis=1)
    a = jnp.dot(patches, w_ref[...], preferred_element_type=jnp.float32)
    a1 = jnp.concatenate([a[1:], a[HW - 1 :]], axis=0)
    m1 = jnp.maximum(a, a1)
    m2 = jnp.maximum(m1[: HW - W], m1[W:])
    mc = jnp.dot(
        s_ref[...], m2.astype(jnp.bfloat16),
        preferred_element_type=jnp.float32,
    )
    o_ref[0] = jnp.maximum(mc + b_ref[...], 0.0).astype(o_ref.dtype)


def _head_body(x_ref, w_ref, b_ref, o_ref):
    o_ref[...] = (
        jnp.dot(x_ref[...], w_ref[...], preferred_element_type=jnp.float32)
        + b_ref[...]
    )


@jax.jit
def kernel(images, conv_w, conv_b, lin_w, lin_b):
    N, C, H, W = images.shape
    HD = conv_w.shape[0]
    K = conv_w.shape[2]
    NCLS = lin_w.shape[0]
    Ho, Wo = H // 2, W // 2
    HWo = Ho * Wo
    G = 4
    NB = N // G
    bf16 = jnp.bfloat16

    x = (
        images.reshape(NB, G, C, H * W)
        .transpose(0, 3, 1, 2)
        .reshape(NB, H * W, G * C)
        .astype(bf16)
    )
    wt = jnp.transpose(conv_w, (2, 3, 1, 0)).reshape(K * K, C, HD)
    eye_g = jnp.eye(G, dtype=wt.dtype)
    w4 = jnp.einsum("gh,tcd->tgchd", eye_g, wt)
    w4 = w4.reshape(K * K * G * C, G * HD).astype(bf16)
    b4 = jnp.tile(conv_b, G).reshape(1, G * HD)
    q = jnp.arange(HWo)
    sel = jax.nn.one_hot(
        2 * W * (q // Wo) + 2 * (q % Wo), H * W - W, dtype=bf16
    )

    pooled = pl.pallas_call(
        functools.partial(
            _conv_pool_body, H=H, W=W, GC=G * C, K=K, GHD=G * HD
        ),
        out_shape=jax.ShapeDtypeStruct((NB, HWo, G * HD), bf16),
        grid=(NB,),
        in_specs=[
            pl.BlockSpec((1, H * W, G * C), lambda n: (n, 0, 0)),
            pl.BlockSpec((K * K * G * C, G * HD), lambda n: (0, 0)),
            pl.BlockSpec((1, G * HD), lambda n: (0, 0)),
            pl.BlockSpec((HWo, H * W - W), lambda n: (0, 0)),
        ],
        out_specs=pl.BlockSpec((1, HWo, G * HD), lambda n: (n, 0, 0)),
        compiler_params=pltpu.CompilerParams(
            dimension_semantics=("parallel",)
        ),
    )(x, w4, b4, sel)

    feats_flat = pooled.reshape(NB, HWo * G * HD)
    wl5 = lin_w.reshape(NCLS, HD, Ho, Wo).transpose(2, 3, 1, 0)
    wl4 = jnp.einsum(
        "gh,ijdc->ijgdhc", jnp.eye(G, dtype=wl5.dtype), wl5
    ).reshape(HWo * G * HD, G * NCLS).astype(bf16)
    bl4 = jnp.tile(lin_b, G).reshape(1, G * NCLS)

    logits4 = pl.pallas_call(
        _head_body,
        out_shape=jax.ShapeDtypeStruct((NB, G * NCLS), jnp.float32),
    )(feats_flat, wl4, bl4)
    return logits4.reshape(N, NCLS)
